# Initial kernel scaffold; baseline (speedup 1.0000x reference)
import jax
import jax.numpy as jnp
from jax.experimental import pallas as pl


def kernel(tensor, batch, num_graphs, weight, bias, mean_scale):
    raise NotImplementedError("write your pallas kernel here")


# speedup vs baseline: 4.5818x; 4.5818x over previous
import functools

import jax
import jax.numpy as jnp
from jax.experimental import pallas as pl

N = 50000
C = 512
G = 128
BLK = 400
NBLK = N // BLK

_INTERPRET = False


def _hi_lo(x):
    hi = x.astype(jnp.bfloat16)
    lo = (x - hi.astype(jnp.float32)).astype(jnp.bfloat16)
    return hi, lo


def _stats_body(x_ref, bf_ref, s1_ref, s2_ref):
    i = pl.program_id(0)

    @pl.when(i == 0)
    def _():
        s1_ref[...] = jnp.zeros_like(s1_ref)
        s2_ref[...] = jnp.zeros_like(s2_ref)

    bvec = bf_ref[...]
    gid = jax.lax.broadcasted_iota(jnp.int32, (BLK, G), 1)
    onehot = (bvec == gid).astype(jnp.bfloat16)

    x = x_ref[...]
    dn = (((0,), (0,)), ((), ()))
    x_hi, x_lo = _hi_lo(x)
    s1_ref[...] += (
        jax.lax.dot_general(onehot, x_hi, dn, preferred_element_type=jnp.float32)
        + jax.lax.dot_general(onehot, x_lo, dn, preferred_element_type=jnp.float32)
    )
    xsq = x * x
    q_hi, q_lo = _hi_lo(xsq)
    s2_ref[...] += (
        jax.lax.dot_general(onehot, q_hi, dn, preferred_element_type=jnp.float32)
        + jax.lax.dot_general(onehot, q_lo, dn, preferred_element_type=jnp.float32)
    )


def _coeff_body(s1_ref, s2_ref, cnt_ref, w_ref, b_ref, ms_ref,
                ah_ref, al_ref, bh_ref, bl_ref):
    inv_c = 1.0 / jnp.maximum(cnt_ref[...], 1.0)
    mean = s1_ref[...] * inv_c
    msub = mean * ms_ref[...]
    var = s2_ref[...] * inv_c - 2.0 * msub * mean + msub * msub
    inv_std = jax.lax.rsqrt(var + 1e-6)
    a = w_ref[...] * inv_std
    b = b_ref[...] - a * msub
    ah, al = _hi_lo(a)
    bh, bl = _hi_lo(b)
    ah_ref[...] = ah
    al_ref[...] = al
    bh_ref[...] = bh
    bl_ref[...] = bl


def _apply_body(x_ref, bf_ref, ah_ref, al_ref, bh_ref, bl_ref, o_ref):
    bvec = bf_ref[...]
    gid = jax.lax.broadcasted_iota(jnp.int32, (BLK, G), 1)
    onehot = (bvec == gid).astype(jnp.bfloat16)
    f32 = jnp.float32
    a = (jnp.dot(onehot, ah_ref[...], preferred_element_type=f32)
         + jnp.dot(onehot, al_ref[...], preferred_element_type=f32))
    b = (jnp.dot(onehot, bh_ref[...], preferred_element_type=f32)
         + jnp.dot(onehot, bl_ref[...], preferred_element_type=f32))
    o_ref[...] = x_ref[...] * a + b


@functools.partial(jax.jit, static_argnums=(2,))
def _graph_norm(tensor, batch, num_graphs, weight, bias, mean_scale):
    del num_graphs
    x = tensor.reshape(N, C)
    bi = batch.astype(jnp.int32)
    batch_f = bi.reshape(N, 1)
    starts = jnp.searchsorted(bi, jnp.arange(G + 1, dtype=jnp.int32))
    counts = (starts[1:] - starts[:-1]).astype(jnp.float32).reshape(G, 1)

    s1, s2 = pl.pallas_call(
        _stats_body,
        grid=(NBLK,),
        in_specs=[
            pl.BlockSpec((BLK, C), lambda i: (i, 0)),
            pl.BlockSpec((BLK, 1), lambda i: (i, 0)),
        ],
        out_specs=[
            pl.BlockSpec((G, C), lambda i: (0, 0)),
            pl.BlockSpec((G, C), lambda i: (0, 0)),
        ],
        out_shape=[
            jax.ShapeDtypeStruct((G, C), jnp.float32),
            jax.ShapeDtypeStruct((G, C), jnp.float32),
        ],
        interpret=_INTERPRET,
    )(x, batch_f)

    w2 = weight.reshape(1, C)
    b2 = bias.reshape(1, C)
    ms2 = mean_scale.reshape(1, C)
    coeff_shape = jax.ShapeDtypeStruct((G, C), jnp.bfloat16)
    ah, al, bh, bl = pl.pallas_call(
        _coeff_body,
        out_shape=[coeff_shape] * 4,
        interpret=_INTERPRET,
    )(s1, s2, counts, w2, b2, ms2)

    out = pl.pallas_call(
        _apply_body,
        grid=(NBLK,),
        in_specs=[
            pl.BlockSpec((BLK, C), lambda i: (i, 0)),
            pl.BlockSpec((BLK, 1), lambda i: (i, 0)),
            pl.BlockSpec((G, C), lambda i: (0, 0)),
            pl.BlockSpec((G, C), lambda i: (0, 0)),
            pl.BlockSpec((G, C), lambda i: (0, 0)),
            pl.BlockSpec((G, C), lambda i: (0, 0)),
        ],
        out_specs=pl.BlockSpec((BLK, C), lambda i: (i, 0)),
        out_shape=jax.ShapeDtypeStruct((N, C), jnp.float32),
        interpret=_INTERPRET,
    )(x, batch_f, ah, al, bh, bl)
    return out.reshape(N, C, 1)


def kernel(tensor, batch, num_graphs, weight, bias, mean_scale):
    return _graph_norm(tensor, batch, G, weight, bias, mean_scale)


# speedup vs baseline: 5.0009x; 1.0915x over previous
import functools

import jax
import jax.numpy as jnp
from jax import lax
from jax.experimental import pallas as pl
from jax.experimental.pallas import tpu as pltpu
from jax.experimental.pallas import tpu_sc as plsc

N = 50000
C = 512
G = 128
NC = 2
NS = 16
NW = NC * NS
GPW = G // NW
CH = 16
NSL = C // 16


def _vext(v16, k):
    lane = lax.iota(jnp.int32, 16)
    return jnp.sum(jnp.where(lane == k, v16, 0))


def _rsqrt_newton(x):
    i = plsc.bitcast(x, jnp.int32)
    y = plsc.bitcast(jnp.int32(0x5F3759DF) - (i >> 1), jnp.float32)
    for _ in range(3):
        y = y * (1.5 - 0.5 * x * y * y)
    return y


def _sc_body(x_hbm, st_hbm, w_hbm, bias_hbm, ms_hbm, out_hbm,
             st_v, buf, obuf, acc1, acc2, av, bv, wv, biasv, msv, sem):
    wid = lax.axis_index("s") * NC + lax.axis_index("c")

    pltpu.sync_copy(st_hbm.at[pl.ds(16 * wid, 16)], st_v)
    pltpu.sync_copy(w_hbm, wv)
    pltpu.sync_copy(bias_hbm, biasv)
    pltpu.sync_copy(ms_hbm, msv)
    s16 = st_v[...]

    for k in range(GPW):
        r0 = s16[k]
        r1 = s16[k + 1]
        nrows = r1 - r0

        for j in range(NSL):
            acc1[pl.ds(16 * j, 16)] = jnp.zeros((16,), jnp.float32)
            acc2[pl.ds(16 * j, 16)] = jnp.zeros((16,), jnp.float32)

        nch = (nrows + CH - 1) >> 4

        def pass_a(ci, _):
            lo = r0 + ci * CH
            base = jnp.minimum(lo, N - CH)
            pltpu.sync_copy(x_hbm.at[pl.ds(base * C, CH * C)], buf)
            fms = []
            for r in range(CH):
                idx = base + r
                ok = jnp.logical_and(idx >= lo, idx < r1)
                fms.append(jnp.where(ok, 1.0, 0.0).astype(jnp.float32))
            for j in range(NSL):
                a1 = acc1[pl.ds(16 * j, 16)]
                a2 = acc2[pl.ds(16 * j, 16)]
                for r in range(CH):
                    x = buf[pl.ds(r * C + 16 * j, 16)]
                    v = x * fms[r]
                    a1 = a1 + v
                    a2 = a2 + v * x
                acc1[pl.ds(16 * j, 16)] = a1
                acc2[pl.ds(16 * j, 16)] = a2
            return 0

        lax.fori_loop(0, nch, pass_a, 0)

        nv = jnp.full((16,), nrows, dtype=jnp.float32)
        inv_n = 1.0 / jnp.maximum(nv, 1.0)
        for j in range(NSL):
            sl = pl.ds(16 * j, 16)
            m = acc1[sl] * inv_n
            ms = m * msv[sl]
            var = acc2[sl] * inv_n - 2.0 * ms * m + ms * ms
            rstd = _rsqrt_newton(var + 1e-6)
            a = wv[sl] * rstd
            av[sl] = a
            bv[sl] = biasv[sl] - a * ms

        big = nrows >= CH

        @pl.when(big)
        def _():
            def pass_b(ci, _):
                base = jnp.minimum(r0 + ci * CH, r1 - CH)
                pltpu.sync_copy(x_hbm.at[pl.ds(base * C, CH * C)], buf)
                for j in range(NSL):
                    a = av[pl.ds(16 * j, 16)]
                    b = bv[pl.ds(16 * j, 16)]
                    for r in range(CH):
                        obuf[pl.ds(r * C + 16 * j, 16)] = (
                            buf[pl.ds(r * C + 16 * j, 16)] * a + b)
                pltpu.sync_copy(obuf, out_hbm.at[pl.ds(base * C, CH * C)])
                return 0

            lax.fori_loop(0, nch, pass_b, 0)

        @pl.when(jnp.logical_and(nrows > 0, jnp.logical_not(big)))
        def _():
            def row_b(r, _):
                row = r0 + r
                pltpu.sync_copy(x_hbm.at[pl.ds(row * C, C)],
                                buf.at[pl.ds(0, C)])
                for j in range(NSL):
                    obuf[pl.ds(16 * j, 16)] = (
                        buf[pl.ds(16 * j, 16)] * av[pl.ds(16 * j, 16)]
                        + bv[pl.ds(16 * j, 16)])
                pltpu.sync_copy(obuf.at[pl.ds(0, C)],
                                out_hbm.at[pl.ds(row * C, C)])
                return 0

            lax.fori_loop(0, nrows, row_b, 0)


@functools.partial(jax.jit, static_argnums=(2,))
def _graph_norm(tensor, batch, num_graphs, weight, bias, mean_scale):
    del num_graphs
    x = tensor.reshape(N * C)
    bi = batch.astype(jnp.int32)
    starts = jnp.searchsorted(bi, jnp.arange(G + 1, dtype=jnp.int32))
    idx = jnp.clip(4 * jnp.arange(NW, dtype=jnp.int32)[:, None]
                   + jnp.arange(16, dtype=jnp.int32)[None, :], 0, G)
    st16 = starts[idx].astype(jnp.int32).reshape(NW * 16)

    mesh = plsc.VectorSubcoreMesh(core_axis_name="c", subcore_axis_name="s",
                                  num_cores=NC, num_subcores=NS)
    run = pl.kernel(
        _sc_body,
        out_type=jax.ShapeDtypeStruct((N * C,), jnp.float32),
        mesh=mesh,
        scratch_types=[
            pltpu.VMEM((16,), jnp.int32),
            pltpu.VMEM((CH * C,), jnp.float32),
            pltpu.VMEM((CH * C,), jnp.float32),
            pltpu.VMEM((C,), jnp.float32),
            pltpu.VMEM((C,), jnp.float32),
            pltpu.VMEM((C,), jnp.float32),
            pltpu.VMEM((C,), jnp.float32),
            pltpu.VMEM((C,), jnp.float32),
            pltpu.VMEM((C,), jnp.float32),
            pltpu.VMEM((C,), jnp.float32),
            pltpu.SemaphoreType.DMA,
        ],
        compiler_params=pltpu.CompilerParams(needs_layout_passes=False),
    )
    out = run(x, st16, weight, bias, mean_scale)
    return out.reshape(N, C, 1)


def kernel(tensor, batch, num_graphs, weight, bias, mean_scale):
    return _graph_norm(tensor, batch, G, weight, bias, mean_scale)


# speedup vs baseline: 6.4226x; 1.2843x over previous
import functools

import jax
import jax.numpy as jnp
from jax import lax
from jax.experimental import pallas as pl
from jax.experimental.pallas import tpu as pltpu
from jax.experimental.pallas import tpu_sc as plsc

N = 50000
C = 512
G = 128
NC = 2
NS = 16
NW = NC * NS
GPW = G // NW
CH = 16
NSL = C // 16


def _rsqrt_newton(x):
    i = plsc.bitcast(x, jnp.int32)
    y = plsc.bitcast(jnp.int32(0x5F3759DF) - (i >> 1), jnp.float32)
    for _ in range(3):
        y = y * (1.5 - 0.5 * x * y * y)
    return y


def _sc_body(x_hbm, st_hbm, w_hbm, bias_hbm, ms_hbm, out_hbm,
             st_v, bufa, bufb, obufa, obufb, acc1, acc2, av, bv, wv, biasv,
             msv, sema, semb, osema, osemb):
    wid = lax.axis_index("s") * NC + lax.axis_index("c")

    pltpu.sync_copy(w_hbm, wv)
    pltpu.sync_copy(bias_hbm, biasv)
    pltpu.sync_copy(ms_hbm, msv)

    def start_in(base, buf, sem):
        pltpu.make_async_copy(
            x_hbm.at[pl.ds(base * C, CH * C)], buf, sem).start()

    def wait_in(buf, sem):
        pltpu.make_async_copy(
            x_hbm.at[pl.ds(0, CH * C)], buf, sem).wait()

    def start_out(obuf, base, osem):
        pltpu.make_async_copy(
            obuf, out_hbm.at[pl.ds(base * C, CH * C)], osem).start()

    def wait_out(obuf, osem):
        pltpu.make_async_copy(
            obuf, out_hbm.at[pl.ds(0, CH * C)], osem).wait()

    def graph_body(kk, _):
        g = wid * GPW + kk
        pltpu.sync_copy(st_hbm.at[pl.ds(g * 16, 16)], st_v)
        s16 = st_v[...]
        r0 = s16[0]
        r1 = s16[1]
        nrows = r1 - r0

        def zero_j(j, _):
            acc1[pl.ds(16 * j, 16)] = jnp.zeros((16,), jnp.float32)
            acc2[pl.ds(16 * j, 16)] = jnp.zeros((16,), jnp.float32)
            return 0

        lax.fori_loop(0, NSL, zero_j, 0)

        nfull = nrows >> 4
        rem = nrows - (nfull << 4)

        def acc_chunk(buf):
            for j in range(NSL):
                a1 = acc1[pl.ds(16 * j, 16)]
                a2 = acc2[pl.ds(16 * j, 16)]
                for r in range(CH):
                    x = buf[pl.ds(r * C + 16 * j, 16)]
                    a1 = a1 + x
                    a2 = a2 + x * x
                acc1[pl.ds(16 * j, 16)] = a1
                acc2[pl.ds(16 * j, 16)] = a2

        @pl.when(nfull > 0)
        def _():
            start_in(r0, bufa, sema)

            def body(ci, _):
                even = (ci & 1) == 0

                @pl.when(even)
                def _():
                    wait_in(bufa, sema)

                    @pl.when(ci + 1 < nfull)
                    def _():
                        start_in(r0 + (ci + 1) * CH, bufb, semb)
                    acc_chunk(bufa)

                @pl.when(jnp.logical_not(even))
                def _():
                    wait_in(bufb, semb)

                    @pl.when(ci + 1 < nfull)
                    def _():
                        start_in(r0 + (ci + 1) * CH, bufa, sema)
                    acc_chunk(bufb)
                return 0

            lax.fori_loop(0, nfull, body, 0)

        @pl.when(rem > 0)
        def _():
            lo = r0 + nfull * CH
            base = jnp.minimum(lo, N - CH)
            shift = lo - base
            pltpu.sync_copy(x_hbm.at[pl.ds(base * C, CH * C)], bufa)

            def tail_r(r, _):
                ok = jnp.logical_and(r >= shift, base + r < r1)
                fm = jnp.full((16,), jnp.where(ok, 1.0, 0.0),
                              dtype=jnp.float32)

                def tail_j(j, _):
                    sl = pl.ds(16 * j, 16)
                    x = bufa[pl.ds(r * C + 16 * j, 16)] * fm
                    acc1[sl] = acc1[sl] + x
                    acc2[sl] = acc2[sl] + x * x
                    return 0

                lax.fori_loop(0, NSL, tail_j, 0)
                return 0

            lax.fori_loop(0, CH, tail_r, 0)

        nv = jnp.full((16,), nrows, dtype=jnp.float32)
        inv_n = 1.0 / jnp.maximum(nv, 1.0)

        def coeff_j(j, _):
            sl = pl.ds(16 * j, 16)
            m = acc1[sl] * inv_n
            ms = m * msv[sl]
            var = acc2[sl] * inv_n - 2.0 * ms * m + ms * ms
            rstd = _rsqrt_newton(var + 1e-6)
            a = wv[sl] * rstd
            av[sl] = a
            bv[sl] = biasv[sl] - a * ms
            return 0

        lax.fori_loop(0, NSL, coeff_j, 0)

        nch = (nrows + CH - 1) >> 4
        big = nrows >= CH

        def apply_chunk(buf, obuf):
            for j in range(NSL):
                a = av[pl.ds(16 * j, 16)]
                b = bv[pl.ds(16 * j, 16)]
                for r in range(CH):
                    obuf[pl.ds(r * C + 16 * j, 16)] = (
                        buf[pl.ds(r * C + 16 * j, 16)] * a + b)

        @pl.when(big)
        def _():
            start_in(r0, bufa, sema)

            def body(ci, _):
                base = jnp.minimum(r0 + ci * CH, r1 - CH)
                nbase = jnp.minimum(r0 + (ci + 1) * CH, r1 - CH)
                even = (ci & 1) == 0

                @pl.when(even)
                def _():
                    wait_in(bufa, sema)

                    @pl.when(ci + 1 < nch)
                    def _():
                        start_in(nbase, bufb, semb)

                    @pl.when(ci >= 2)
                    def _():
                        wait_out(obufa, osema)
                    apply_chunk(bufa, obufa)
                    start_out(obufa, base, osema)

                @pl.when(jnp.logical_not(even))
                def _():
                    wait_in(bufb, semb)

                    @pl.when(ci + 1 < nch)
                    def _():
                        start_in(nbase, bufa, sema)

                    @pl.when(ci >= 2)
                    def _():
                        wait_out(obufb, osemb)
                    apply_chunk(bufb, obufb)
                    start_out(obufb, base, osemb)
                return 0

            lax.fori_loop(0, nch, body, 0)

            last_even = ((nch - 1) & 1) == 0

            @pl.when(last_even)
            def _():
                wait_out(obufa, osema)

            @pl.when(jnp.logical_not(last_even))
            def _():
                wait_out(obufb, osemb)

            @pl.when(jnp.logical_and(nch >= 2, last_even))
            def _():
                wait_out(obufb, osemb)

            @pl.when(jnp.logical_and(nch >= 2, jnp.logical_not(last_even)))
            def _():
                wait_out(obufa, osema)

        @pl.when(jnp.logical_and(nrows > 0, jnp.logical_not(big)))
        def _():
            def row_b(r, _):
                row = r0 + r
                pltpu.sync_copy(x_hbm.at[pl.ds(row * C, C)],
                                bufa.at[pl.ds(0, C)])

                def row_j(j, _):
                    obufa[pl.ds(16 * j, 16)] = (
                        bufa[pl.ds(16 * j, 16)] * av[pl.ds(16 * j, 16)]
                        + bv[pl.ds(16 * j, 16)])
                    return 0

                lax.fori_loop(0, NSL, row_j, 0)
                pltpu.sync_copy(obufa.at[pl.ds(0, C)],
                                out_hbm.at[pl.ds(row * C, C)])
                return 0

            lax.fori_loop(0, nrows, row_b, 0)
        return 0

    lax.fori_loop(0, GPW, graph_body, 0)


@functools.partial(jax.jit, static_argnums=(2,))
def _graph_norm(tensor, batch, num_graphs, weight, bias, mean_scale):
    del num_graphs
    x = tensor.reshape(N * C)
    bi = batch.astype(jnp.int32)
    starts = jnp.searchsorted(bi, jnp.arange(G + 1, dtype=jnp.int32))
    st16 = jnp.zeros((G, 16), jnp.int32)
    st16 = st16.at[:, 0].set(starts[:-1]).at[:, 1].set(starts[1:])
    st16 = st16.reshape(G * 16)

    mesh = plsc.VectorSubcoreMesh(core_axis_name="c", subcore_axis_name="s",
                                  num_cores=NC, num_subcores=NS)
    run = pl.kernel(
        _sc_body,
        out_type=jax.ShapeDtypeStruct((N * C,), jnp.float32),
        mesh=mesh,
        scratch_types=[
            pltpu.VMEM((16,), jnp.int32),
            pltpu.VMEM((CH * C,), jnp.float32),
            pltpu.VMEM((CH * C,), jnp.float32),
            pltpu.VMEM((CH * C,), jnp.float32),
            pltpu.VMEM((CH * C,), jnp.float32),
            pltpu.VMEM((C,), jnp.float32),
            pltpu.VMEM((C,), jnp.float32),
            pltpu.VMEM((C,), jnp.float32),
            pltpu.VMEM((C,), jnp.float32),
            pltpu.VMEM((C,), jnp.float32),
            pltpu.VMEM((C,), jnp.float32),
            pltpu.VMEM((C,), jnp.float32),
            pltpu.SemaphoreType.DMA,
            pltpu.SemaphoreType.DMA,
            pltpu.SemaphoreType.DMA,
            pltpu.SemaphoreType.DMA,
        ],
        compiler_params=pltpu.CompilerParams(needs_layout_passes=False),
    )
    out = run(x, st16, weight, bias, mean_scale)
    return out.reshape(N, C, 1)


def kernel(tensor, batch, num_graphs, weight, bias, mean_scale):
    return _graph_norm(tensor, batch, G, weight, bias, mean_scale)


# speedup vs baseline: 7.6361x; 1.1890x over previous
import functools

import jax
import jax.numpy as jnp
from jax import lax
from jax.experimental import pallas as pl
from jax.experimental.pallas import tpu as pltpu
from jax.experimental.pallas import tpu_sc as plsc

N = 50000
C = 512
G = 128
NC = 2
NS = 16
NW = NC * NS
GPW = G // NW
CH = 32
CHS = 5
NSL = C // 16


def _rsqrt_newton(x):
    i = plsc.bitcast(x, jnp.int32)
    y = plsc.bitcast(jnp.int32(0x5F3759DF) - (i >> 1), jnp.float32)
    for _ in range(3):
        y = y * (1.5 - 0.5 * x * y * y)
    return y


def _sc_body(x_hbm, st_hbm, w_hbm, bias_hbm, ms_hbm, out_hbm,
             st_v, bufa, bufb, obufa, obufb, acc1, acc2, av, bv, wv, biasv,
             msv, sema, semb, osema, osemb):
    wid = lax.axis_index("s") * NC + lax.axis_index("c")

    pltpu.sync_copy(w_hbm, wv)
    pltpu.sync_copy(bias_hbm, biasv)
    pltpu.sync_copy(ms_hbm, msv)

    def start_in(base, buf, sem):
        pltpu.make_async_copy(
            x_hbm.at[pl.ds(base * C, CH * C)], buf, sem).start()

    def wait_in(buf, sem):
        pltpu.make_async_copy(
            x_hbm.at[pl.ds(0, CH * C)], buf, sem).wait()

    def start_out(obuf, base, osem):
        pltpu.make_async_copy(
            obuf, out_hbm.at[pl.ds(base * C, CH * C)], osem).start()

    def wait_out(obuf, osem):
        pltpu.make_async_copy(
            obuf, out_hbm.at[pl.ds(0, CH * C)], osem).wait()

    def graph_body(kk, _):
        g = wid * GPW + kk
        pltpu.sync_copy(st_hbm.at[pl.ds(g * 16, 16)], st_v)
        s16 = st_v[...]
        r0 = s16[0]
        r1 = s16[1]
        nrows = r1 - r0

        def zero_j(j, _):
            acc1[pl.ds(16 * j, 16)] = jnp.zeros((16,), jnp.float32)
            acc2[pl.ds(16 * j, 16)] = jnp.zeros((16,), jnp.float32)
            return 0

        lax.fori_loop(0, NSL, zero_j, 0)

        nfull = nrows >> CHS
        rem = nrows - (nfull << CHS)

        def acc_chunk(buf):
            def accj(j, _):
                off = 16 * j
                a1 = acc1[pl.ds(off, 16)]
                a2 = acc2[pl.ds(off, 16)]
                for r in range(CH):
                    x = buf[pl.ds(r * C + off, 16)]
                    a1 = a1 + x
                    a2 = a2 + x * x
                acc1[pl.ds(off, 16)] = a1
                acc2[pl.ds(off, 16)] = a2
                return 0

            lax.fori_loop(0, NSL, accj, 0)

        @pl.when(nfull > 0)
        def _():
            start_in(r0, bufa, sema)

            def body(ci, _):
                even = (ci & 1) == 0

                @pl.when(even)
                def _():
                    wait_in(bufa, sema)

                    @pl.when(ci + 1 < nfull)
                    def _():
                        start_in(r0 + (ci + 1) * CH, bufb, semb)
                    acc_chunk(bufa)

                @pl.when(jnp.logical_not(even))
                def _():
                    wait_in(bufb, semb)

                    @pl.when(ci + 1 < nfull)
                    def _():
                        start_in(r0 + (ci + 1) * CH, bufa, sema)
                    acc_chunk(bufb)
                return 0

            lax.fori_loop(0, nfull, body, 0)

        @pl.when(rem > 0)
        def _():
            lo = r0 + nfull * CH
            base = jnp.minimum(lo, N - CH)
            shift = lo - base
            pltpu.sync_copy(x_hbm.at[pl.ds(base * C, CH * C)], bufa)

            def tail_r(r, _):
                ok = jnp.logical_and(r >= shift, base + r < r1)
                fm = jnp.full((16,), jnp.where(ok, 1.0, 0.0),
                              dtype=jnp.float32)

                def tail_j(j, _):
                    sl = pl.ds(16 * j, 16)
                    x = bufa[pl.ds(r * C + 16 * j, 16)] * fm
                    acc1[sl] = acc1[sl] + x
                    acc2[sl] = acc2[sl] + x * x
                    return 0

                lax.fori_loop(0, NSL, tail_j, 0)
                return 0

            lax.fori_loop(0, CH, tail_r, 0)

        nv = jnp.full((16,), nrows, dtype=jnp.float32)
        inv_n = 1.0 / jnp.maximum(nv, 1.0)

        def coeff_j(j, _):
            sl = pl.ds(16 * j, 16)
            m = acc1[sl] * inv_n
            ms = m * msv[sl]
            var = acc2[sl] * inv_n - 2.0 * ms * m + ms * ms
            rstd = _rsqrt_newton(var + 1e-6)
            a = wv[sl] * rstd
            av[sl] = a
            bv[sl] = biasv[sl] - a * ms
            return 0

        lax.fori_loop(0, NSL, coeff_j, 0)

        nch = (nrows + CH - 1) >> CHS
        big = nrows >= CH

        def apply_chunk(buf, obuf):
            def appj(j, _):
                off = 16 * j
                a = av[pl.ds(off, 16)]
                b = bv[pl.ds(off, 16)]
                for r in range(CH):
                    obuf[pl.ds(r * C + off, 16)] = (
                        buf[pl.ds(r * C + off, 16)] * a + b)
                return 0

            lax.fori_loop(0, NSL, appj, 0)

        @pl.when(big)
        def _():
            start_in(r0, bufa, sema)

            def body(ci, _):
                base = jnp.minimum(r0 + ci * CH, r1 - CH)
                nbase = jnp.minimum(r0 + (ci + 1) * CH, r1 - CH)
                even = (ci & 1) == 0

                @pl.when(even)
                def _():
                    wait_in(bufa, sema)

                    @pl.when(ci + 1 < nch)
                    def _():
                        start_in(nbase, bufb, semb)

                    @pl.when(ci >= 2)
                    def _():
                        wait_out(obufa, osema)
                    apply_chunk(bufa, obufa)
                    start_out(obufa, base, osema)

                @pl.when(jnp.logical_not(even))
                def _():
                    wait_in(bufb, semb)

                    @pl.when(ci + 1 < nch)
                    def _():
                        start_in(nbase, bufa, sema)

                    @pl.when(ci >= 2)
                    def _():
                        wait_out(obufb, osemb)
                    apply_chunk(bufb, obufb)
                    start_out(obufb, base, osemb)
                return 0

            lax.fori_loop(0, nch, body, 0)

            last_even = ((nch - 1) & 1) == 0

            @pl.when(last_even)
            def _():
                wait_out(obufa, osema)

            @pl.when(jnp.logical_not(last_even))
            def _():
                wait_out(obufb, osemb)

            @pl.when(jnp.logical_and(nch >= 2, last_even))
            def _():
                wait_out(obufb, osemb)

            @pl.when(jnp.logical_and(nch >= 2, jnp.logical_not(last_even)))
            def _():
                wait_out(obufa, osema)

        @pl.when(jnp.logical_and(nrows > 0, jnp.logical_not(big)))
        def _():
            def row_b(r, _):
                row = r0 + r
                pltpu.sync_copy(x_hbm.at[pl.ds(row * C, C)],
                                bufa.at[pl.ds(0, C)])

                def row_j(j, _):
                    obufa[pl.ds(16 * j, 16)] = (
                        bufa[pl.ds(16 * j, 16)] * av[pl.ds(16 * j, 16)]
                        + bv[pl.ds(16 * j, 16)])
                    return 0

                lax.fori_loop(0, NSL, row_j, 0)
                pltpu.sync_copy(obufa.at[pl.ds(0, C)],
                                out_hbm.at[pl.ds(row * C, C)])
                return 0

            lax.fori_loop(0, nrows, row_b, 0)
        return 0

    lax.fori_loop(0, GPW, graph_body, 0)


@functools.partial(jax.jit, static_argnums=(2,))
def _graph_norm(tensor, batch, num_graphs, weight, bias, mean_scale):
    del num_graphs
    x = tensor.reshape(N * C)
    bi = batch.astype(jnp.int32)
    starts = jnp.searchsorted(bi, jnp.arange(G + 1, dtype=jnp.int32),
                              method="scan_unrolled")
    st16 = jnp.zeros((G, 16), jnp.int32)
    st16 = st16.at[:, 0].set(starts[:-1]).at[:, 1].set(starts[1:])
    st16 = st16.reshape(G * 16)

    mesh = plsc.VectorSubcoreMesh(core_axis_name="c", subcore_axis_name="s",
                                  num_cores=NC, num_subcores=NS)
    run = pl.kernel(
        _sc_body,
        out_type=jax.ShapeDtypeStruct((N * C,), jnp.float32),
        mesh=mesh,
        scratch_types=[
            pltpu.VMEM((16,), jnp.int32),
            pltpu.VMEM((CH * C,), jnp.float32),
            pltpu.VMEM((CH * C,), jnp.float32),
            pltpu.VMEM((CH * C,), jnp.float32),
            pltpu.VMEM((CH * C,), jnp.float32),
            pltpu.VMEM((C,), jnp.float32),
            pltpu.VMEM((C,), jnp.float32),
            pltpu.VMEM((C,), jnp.float32),
            pltpu.VMEM((C,), jnp.float32),
            pltpu.VMEM((C,), jnp.float32),
            pltpu.VMEM((C,), jnp.float32),
            pltpu.VMEM((C,), jnp.float32),
            pltpu.SemaphoreType.DMA,
            pltpu.SemaphoreType.DMA,
            pltpu.SemaphoreType.DMA,
            pltpu.SemaphoreType.DMA,
        ],
        compiler_params=pltpu.CompilerParams(needs_layout_passes=False),
    )
    out = run(x, st16, weight, bias, mean_scale)
    return out.reshape(N, C, 1)


def kernel(tensor, batch, num_graphs, weight, bias, mean_scale):
    return _graph_norm(tensor, batch, G, weight, bias, mean_scale)


# speedup vs baseline: 8.2399x; 1.0791x over previous
import functools

import jax
import jax.numpy as jnp
from jax import lax
from jax.experimental import pallas as pl
from jax.experimental.pallas import tpu as pltpu
from jax.experimental.pallas import tpu_sc as plsc

N = 50000
C = 512
G = 128
NC = 2
NS = 16
NW = NC * NS
GPW = G // NW
CH = 32
CHS = 5
NSL = C // 16


def _rsqrt_newton(x):
    i = plsc.bitcast(x, jnp.int32)
    y = plsc.bitcast(jnp.int32(0x5F3759DF) - (i >> 1), jnp.float32)
    for _ in range(3):
        y = y * (1.5 - 0.5 * x * y * y)
    return y


def _sc_body(x_hbm, st_hbm, w_hbm, bias_hbm, ms_hbm, out_hbm,
             st_v, bufa, bufb, obufa, obufb, acc1, acc2, av, bv, wv, biasv,
             msv, sema, semb, osema, osemb):
    wid = lax.axis_index("s") * NC + lax.axis_index("c")

    pltpu.sync_copy(w_hbm, wv)
    pltpu.sync_copy(bias_hbm, biasv)
    pltpu.sync_copy(ms_hbm, msv)

    def start_in(base, buf, sem):
        pltpu.make_async_copy(
            x_hbm.at[pl.ds(base * C, CH * C)], buf, sem).start()

    def wait_in(buf, sem):
        pltpu.make_async_copy(
            x_hbm.at[pl.ds(0, CH * C)], buf, sem).wait()

    def start_out(obuf, base, osem):
        pltpu.make_async_copy(
            obuf, out_hbm.at[pl.ds(base * C, CH * C)], osem).start()

    def wait_out(obuf, osem):
        pltpu.make_async_copy(
            obuf, out_hbm.at[pl.ds(0, CH * C)], osem).wait()

    def graph_body(kk, _):
        g = wid * GPW + kk
        pltpu.sync_copy(st_hbm.at[pl.ds(g * 16, 16)], st_v)
        s16 = st_v[...]
        r0 = s16[0]
        r1 = s16[1]
        nrows = r1 - r0

        def zero_j(j, _):
            acc1[pl.ds(16 * j, 16)] = jnp.zeros((16,), jnp.float32)
            acc2[pl.ds(16 * j, 16)] = jnp.zeros((16,), jnp.float32)
            return 0

        lax.fori_loop(0, NSL, zero_j, 0)

        nfull = nrows >> CHS
        rem = nrows - (nfull << CHS)

        def acc_chunk(buf):
            def accj(j, _):
                off = 16 * j
                a1 = acc1[pl.ds(off, 16)]
                a2 = acc2[pl.ds(off, 16)]
                for r in range(CH):
                    x = buf[pl.ds(r * C + off, 16)]
                    a1 = a1 + x
                    a2 = a2 + x * x
                acc1[pl.ds(off, 16)] = a1
                acc2[pl.ds(off, 16)] = a2
                return 0

            lax.fori_loop(0, NSL, accj, 0)

        @pl.when(nfull > 0)
        def _():
            start_in(r0, bufa, sema)

            def body(ci, _):
                even = (ci & 1) == 0

                @pl.when(even)
                def _():
                    wait_in(bufa, sema)

                    @pl.when(ci + 1 < nfull)
                    def _():
                        start_in(r0 + (ci + 1) * CH, bufb, semb)
                    acc_chunk(bufa)

                @pl.when(jnp.logical_not(even))
                def _():
                    wait_in(bufb, semb)

                    @pl.when(ci + 1 < nfull)
                    def _():
                        start_in(r0 + (ci + 1) * CH, bufa, sema)
                    acc_chunk(bufb)
                return 0

            lax.fori_loop(0, nfull, body, 0)

        @pl.when(rem > 0)
        def _():
            lo = r0 + nfull * CH
            base = jnp.minimum(lo, N - CH)
            shift = lo - base
            pltpu.sync_copy(x_hbm.at[pl.ds(base * C, CH * C)], bufa)

            def tail_r(r, _):
                ok = jnp.logical_and(r >= shift, base + r < r1)
                fm = jnp.full((16,), jnp.where(ok, 1.0, 0.0),
                              dtype=jnp.float32)

                def tail_j(j, _):
                    sl = pl.ds(16 * j, 16)
                    x = bufa[pl.ds(r * C + 16 * j, 16)] * fm
                    acc1[sl] = acc1[sl] + x
                    acc2[sl] = acc2[sl] + x * x
                    return 0

                lax.fori_loop(0, NSL, tail_j, 0)
                return 0

            lax.fori_loop(0, CH, tail_r, 0)

        nv = jnp.full((16,), nrows, dtype=jnp.float32)
        inv_n = 1.0 / jnp.maximum(nv, 1.0)

        def coeff_j(j, _):
            sl = pl.ds(16 * j, 16)
            m = acc1[sl] * inv_n
            ms = m * msv[sl]
            var = acc2[sl] * inv_n - 2.0 * ms * m + ms * ms
            rstd = _rsqrt_newton(var + 1e-6)
            a = wv[sl] * rstd
            av[sl] = a
            bv[sl] = biasv[sl] - a * ms
            return 0

        lax.fori_loop(0, NSL, coeff_j, 0)

        nch = (nrows + CH - 1) >> CHS
        big = nrows >= CH

        def apply_chunk(buf, obuf):
            def appj(j, _):
                off = 16 * j
                a = av[pl.ds(off, 16)]
                b = bv[pl.ds(off, 16)]
                for r in range(CH):
                    obuf[pl.ds(r * C + off, 16)] = (
                        buf[pl.ds(r * C + off, 16)] * a + b)
                return 0

            lax.fori_loop(0, NSL, appj, 0)

        @pl.when(big)
        def _():
            start_in(r0, bufa, sema)

            def body(ci, _):
                base = jnp.minimum(r0 + ci * CH, r1 - CH)
                nbase = jnp.minimum(r0 + (ci + 1) * CH, r1 - CH)
                even = (ci & 1) == 0

                @pl.when(even)
                def _():
                    wait_in(bufa, sema)

                    @pl.when(ci + 1 < nch)
                    def _():
                        start_in(nbase, bufb, semb)

                    @pl.when(ci >= 2)
                    def _():
                        wait_out(obufa, osema)
                    apply_chunk(bufa, obufa)
                    start_out(obufa, base, osema)

                @pl.when(jnp.logical_not(even))
                def _():
                    wait_in(bufb, semb)

                    @pl.when(ci + 1 < nch)
                    def _():
                        start_in(nbase, bufa, sema)

                    @pl.when(ci >= 2)
                    def _():
                        wait_out(obufb, osemb)
                    apply_chunk(bufb, obufb)
                    start_out(obufb, base, osemb)
                return 0

            lax.fori_loop(0, nch, body, 0)

            last_even = ((nch - 1) & 1) == 0

            @pl.when(last_even)
            def _():
                wait_out(obufa, osema)

            @pl.when(jnp.logical_not(last_even))
            def _():
                wait_out(obufb, osemb)

            @pl.when(jnp.logical_and(nch >= 2, last_even))
            def _():
                wait_out(obufb, osemb)

            @pl.when(jnp.logical_and(nch >= 2, jnp.logical_not(last_even)))
            def _():
                wait_out(obufa, osema)

        @pl.when(jnp.logical_and(nrows > 0, jnp.logical_not(big)))
        def _():
            def row_b(r, _):
                row = r0 + r
                pltpu.sync_copy(x_hbm.at[pl.ds(row * C, C)],
                                bufa.at[pl.ds(0, C)])

                def row_j(j, _):
                    obufa[pl.ds(16 * j, 16)] = (
                        bufa[pl.ds(16 * j, 16)] * av[pl.ds(16 * j, 16)]
                        + bv[pl.ds(16 * j, 16)])
                    return 0

                lax.fori_loop(0, NSL, row_j, 0)
                pltpu.sync_copy(obufa.at[pl.ds(0, C)],
                                out_hbm.at[pl.ds(row * C, C)])
                return 0

            lax.fori_loop(0, nrows, row_b, 0)
        return 0

    lax.fori_loop(0, GPW, graph_body, 0)


@functools.partial(jax.jit, static_argnums=(2,))
def _graph_norm(tensor, batch, num_graphs, weight, bias, mean_scale):
    del num_graphs
    x = tensor.reshape(N * C)
    bi = batch.astype(jnp.int32)
    starts = jnp.searchsorted(bi, jnp.arange(G + 1, dtype=jnp.int32),
                              method="compare_all")
    st16 = jnp.zeros((G, 16), jnp.int32)
    st16 = st16.at[:, 0].set(starts[:-1]).at[:, 1].set(starts[1:])
    st16 = st16.reshape(G * 16)

    mesh = plsc.VectorSubcoreMesh(core_axis_name="c", subcore_axis_name="s",
                                  num_cores=NC, num_subcores=NS)
    run = pl.kernel(
        _sc_body,
        out_type=jax.ShapeDtypeStruct((N * C,), jnp.float32),
        mesh=mesh,
        scratch_types=[
            pltpu.VMEM((16,), jnp.int32),
            pltpu.VMEM((CH * C,), jnp.float32),
            pltpu.VMEM((CH * C,), jnp.float32),
            pltpu.VMEM((CH * C,), jnp.float32),
            pltpu.VMEM((CH * C,), jnp.float32),
            pltpu.VMEM((C,), jnp.float32),
            pltpu.VMEM((C,), jnp.float32),
            pltpu.VMEM((C,), jnp.float32),
            pltpu.VMEM((C,), jnp.float32),
            pltpu.VMEM((C,), jnp.float32),
            pltpu.VMEM((C,), jnp.float32),
            pltpu.VMEM((C,), jnp.float32),
            pltpu.SemaphoreType.DMA,
            pltpu.SemaphoreType.DMA,
            pltpu.SemaphoreType.DMA,
            pltpu.SemaphoreType.DMA,
        ],
        compiler_params=pltpu.CompilerParams(needs_layout_passes=False),
    )
    out = run(x, st16, weight, bias, mean_scale)
    return out.reshape(N, C, 1)


def kernel(tensor, batch, num_graphs, weight, bias, mean_scale):
    return _graph_norm(tensor, batch, G, weight, bias, mean_scale)


# speedup vs baseline: 8.7285x; 1.0593x over previous
import functools

import jax
import jax.numpy as jnp
from jax import lax
from jax.experimental import pallas as pl
from jax.experimental.pallas import tpu as pltpu
from jax.experimental.pallas import tpu_sc as plsc

N = 50000
C = 512
G = 128
NC = 2
NS = 16
NW = NC * NS
GPW = G // NW
CH = 32
CHS = 5
NSL = C // 16


def _rsqrt_newton(x):
    i = plsc.bitcast(x, jnp.int32)
    y = plsc.bitcast(jnp.int32(0x5F3759DF) - (i >> 1), jnp.float32)
    for _ in range(3):
        y = y * (1.5 - 0.5 * x * y * y)
    return y


def _sc_body(x_hbm, st_hbm, w_hbm, bias_hbm, ms_hbm, out_hbm,
             st_v, bufa, bufb, bufc, obufa, obufb, acc1, acc2, av, bv, wv,
             biasv, msv, sema, semb, semc, osema, osemb):
    wid = lax.axis_index("s") * NC + lax.axis_index("c")

    pltpu.sync_copy(w_hbm, wv)
    pltpu.sync_copy(bias_hbm, biasv)
    pltpu.sync_copy(ms_hbm, msv)

    def start_in(base, buf, sem):
        pltpu.make_async_copy(
            x_hbm.at[pl.ds(base * C, CH * C)], buf, sem).start()

    def wait_in(buf, sem):
        pltpu.make_async_copy(
            x_hbm.at[pl.ds(0, CH * C)], buf, sem).wait()

    def start_out(obuf, base, osem):
        pltpu.make_async_copy(
            obuf, out_hbm.at[pl.ds(base * C, CH * C)], osem).start()

    def wait_out(obuf, osem):
        pltpu.make_async_copy(
            obuf, out_hbm.at[pl.ds(0, CH * C)], osem).wait()

    def graph_body(kk, _):
        g = wid * GPW + kk
        pltpu.sync_copy(st_hbm.at[pl.ds(g * 16, 16)], st_v)
        s16 = st_v[...]
        r0 = s16[0]
        r1 = s16[1]
        nrows = r1 - r0

        def zero_j(j, _):
            acc1[pl.ds(16 * j, 16)] = jnp.zeros((16,), jnp.float32)
            acc2[pl.ds(16 * j, 16)] = jnp.zeros((16,), jnp.float32)
            return 0

        lax.fori_loop(0, NSL, zero_j, 0)

        nfull = nrows >> CHS
        rem = nrows - (nfull << CHS)

        def acc_chunk(buf):
            def accj(j, _):
                off = 16 * j
                a1 = acc1[pl.ds(off, 16)]
                a2 = acc2[pl.ds(off, 16)]
                for r in range(CH):
                    x = buf[pl.ds(r * C + off, 16)]
                    a1 = a1 + x
                    a2 = a2 + x * x
                acc1[pl.ds(off, 16)] = a1
                acc2[pl.ds(off, 16)] = a2
                return 0

            lax.fori_loop(0, NSL, accj, 0)

        bufs = (bufa, bufb, bufc)
        sems = (sema, semb, semc)

        @pl.when(nfull > 0)
        def _():
            start_in(r0, bufa, sema)

            @pl.when(nfull > 1)
            def _():
                start_in(r0 + CH, bufb, semb)

            def body(ci, p):
                for q in range(3):
                    def br(q=q):
                        wait_in(bufs[q], sems[q])

                        @pl.when(ci + 2 < nfull)
                        def _():
                            start_in(r0 + (ci + 2) * CH,
                                     bufs[(q + 2) % 3], sems[(q + 2) % 3])
                        acc_chunk(bufs[q])
                    pl.when(p == q)(br)
                return jnp.where(p == 2, 0, p + 1)

            lax.fori_loop(0, nfull, body, jnp.int32(0))

        @pl.when(rem > 0)
        def _():
            lo = r0 + nfull * CH
            base = jnp.minimum(lo, N - CH)
            shift = lo - base
            pltpu.sync_copy(x_hbm.at[pl.ds(base * C, CH * C)], bufa)

            def tail_r(r, _):
                ok = jnp.logical_and(r >= shift, base + r < r1)
                fm = jnp.full((16,), jnp.where(ok, 1.0, 0.0),
                              dtype=jnp.float32)

                def tail_j(j, _):
                    sl = pl.ds(16 * j, 16)
                    x = bufa[pl.ds(r * C + 16 * j, 16)] * fm
                    acc1[sl] = acc1[sl] + x
                    acc2[sl] = acc2[sl] + x * x
                    return 0

                lax.fori_loop(0, NSL, tail_j, 0)
                return 0

            lax.fori_loop(0, CH, tail_r, 0)

        nv = jnp.full((16,), nrows, dtype=jnp.float32)
        inv_n = 1.0 / jnp.maximum(nv, 1.0)

        def coeff_j(j, _):
            sl = pl.ds(16 * j, 16)
            m = acc1[sl] * inv_n
            ms = m * msv[sl]
            var = acc2[sl] * inv_n - 2.0 * ms * m + ms * ms
            rstd = _rsqrt_newton(var + 1e-6)
            a = wv[sl] * rstd
            av[sl] = a
            bv[sl] = biasv[sl] - a * ms
            return 0

        lax.fori_loop(0, NSL, coeff_j, 0)

        nch = (nrows + CH - 1) >> CHS
        big = nrows >= CH

        def apply_chunk(buf, obuf):
            def appj(j, _):
                off = 16 * j
                a = av[pl.ds(off, 16)]
                b = bv[pl.ds(off, 16)]
                for r in range(CH):
                    obuf[pl.ds(r * C + off, 16)] = (
                        buf[pl.ds(r * C + off, 16)] * a + b)
                return 0

            lax.fori_loop(0, NSL, appj, 0)

        @pl.when(big)
        def _():
            start_in(r0, bufa, sema)

            @pl.when(nch > 1)
            def _():
                start_in(jnp.minimum(r0 + CH, r1 - CH), bufb, semb)

            def body(ci, p):
                base = jnp.minimum(r0 + ci * CH, r1 - CH)
                nbase2 = jnp.minimum(r0 + (ci + 2) * CH, r1 - CH)
                even = (ci & 1) == 0
                for q in range(3):
                    def br(q=q):
                        wait_in(bufs[q], sems[q])

                        @pl.when(ci + 2 < nch)
                        def _():
                            start_in(nbase2, bufs[(q + 2) % 3],
                                     sems[(q + 2) % 3])

                        @pl.when(even)
                        def _():
                            @pl.when(ci >= 2)
                            def _():
                                wait_out(obufa, osema)
                            apply_chunk(bufs[q], obufa)
                            start_out(obufa, base, osema)

                        @pl.when(jnp.logical_not(even))
                        def _():
                            @pl.when(ci >= 2)
                            def _():
                                wait_out(obufb, osemb)
                            apply_chunk(bufs[q], obufb)
                            start_out(obufb, base, osemb)
                    pl.when(p == q)(br)
                return jnp.where(p == 2, 0, p + 1)

            lax.fori_loop(0, nch, body, jnp.int32(0))

            last_even = ((nch - 1) & 1) == 0

            @pl.when(last_even)
            def _():
                wait_out(obufa, osema)

            @pl.when(jnp.logical_not(last_even))
            def _():
                wait_out(obufb, osemb)

            @pl.when(jnp.logical_and(nch >= 2, last_even))
            def _():
                wait_out(obufb, osemb)

            @pl.when(jnp.logical_and(nch >= 2, jnp.logical_not(last_even)))
            def _():
                wait_out(obufa, osema)

        @pl.when(jnp.logical_and(nrows > 0, jnp.logical_not(big)))
        def _():
            def row_b(r, _):
                row = r0 + r
                pltpu.sync_copy(x_hbm.at[pl.ds(row * C, C)],
                                bufa.at[pl.ds(0, C)])

                def row_j(j, _):
                    obufa[pl.ds(16 * j, 16)] = (
                        bufa[pl.ds(16 * j, 16)] * av[pl.ds(16 * j, 16)]
                        + bv[pl.ds(16 * j, 16)])
                    return 0

                lax.fori_loop(0, NSL, row_j, 0)
                pltpu.sync_copy(obufa.at[pl.ds(0, C)],
                                out_hbm.at[pl.ds(row * C, C)])
                return 0

            lax.fori_loop(0, nrows, row_b, 0)
        return 0

    lax.fori_loop(0, GPW, graph_body, 0)


@functools.partial(jax.jit, static_argnums=(2,))
def _graph_norm(tensor, batch, num_graphs, weight, bias, mean_scale):
    del num_graphs
    x = tensor.reshape(N * C)
    bi = batch.astype(jnp.int32)
    starts = jnp.searchsorted(bi, jnp.arange(G + 1, dtype=jnp.int32),
                              method="compare_all")
    st16 = jnp.zeros((G, 16), jnp.int32)
    st16 = st16.at[:, 0].set(starts[:-1]).at[:, 1].set(starts[1:])
    st16 = st16.reshape(G * 16)

    mesh = plsc.VectorSubcoreMesh(core_axis_name="c", subcore_axis_name="s",
                                  num_cores=NC, num_subcores=NS)
    run = pl.kernel(
        _sc_body,
        out_type=jax.ShapeDtypeStruct((N * C,), jnp.float32),
        mesh=mesh,
        scratch_types=[
            pltpu.VMEM((16,), jnp.int32),
            pltpu.VMEM((CH * C,), jnp.float32),
            pltpu.VMEM((CH * C,), jnp.float32),
            pltpu.VMEM((CH * C,), jnp.float32),
            pltpu.VMEM((CH * C,), jnp.float32),
            pltpu.VMEM((CH * C,), jnp.float32),
            pltpu.VMEM((C,), jnp.float32),
            pltpu.VMEM((C,), jnp.float32),
            pltpu.VMEM((C,), jnp.float32),
            pltpu.VMEM((C,), jnp.float32),
            pltpu.VMEM((C,), jnp.float32),
            pltpu.VMEM((C,), jnp.float32),
            pltpu.VMEM((C,), jnp.float32),
            pltpu.SemaphoreType.DMA,
            pltpu.SemaphoreType.DMA,
            pltpu.SemaphoreType.DMA,
            pltpu.SemaphoreType.DMA,
            pltpu.SemaphoreType.DMA,
        ],
        compiler_params=pltpu.CompilerParams(needs_layout_passes=False),
    )
    out = run(x, st16, weight, bias, mean_scale)
    return out.reshape(N, C, 1)


def kernel(tensor, batch, num_graphs, weight, bias, mean_scale):
    return _graph_norm(tensor, batch, G, weight, bias, mean_scale)


# speedup vs baseline: 23.0644x; 2.6424x over previous
import functools

import jax
import jax.numpy as jnp
from jax import lax
from jax.experimental import pallas as pl
from jax.experimental.pallas import tpu as pltpu
from jax.experimental.pallas import tpu_sc as plsc

N = 50000
C = 512
G = 128
NC = 2
NS = 16
NW = NC * NS
GPW = G // NW
CH = 32
CHS = 5
NSL = C // 16


def _rsqrt_newton(x):
    i = plsc.bitcast(x, jnp.int32)
    y = plsc.bitcast(jnp.int32(0x5F3759DF) - (i >> 1), jnp.float32)
    for _ in range(3):
        y = y * (1.5 - 0.5 * x * y * y)
    return y


def _sc_body(x_hbm, st_hbm, w_hbm, bias_hbm, ms_hbm, out_hbm,
             st_v, bufa, bufb, bufc, obufa, obufb, acc1, acc2, av, bv, wv,
             biasv, msv, sema, semb, semc, osema, osemb):
    wid = lax.axis_index("s") * NC + lax.axis_index("c")

    pltpu.sync_copy(w_hbm, wv)
    pltpu.sync_copy(bias_hbm, biasv)
    pltpu.sync_copy(ms_hbm, msv)

    def start_in(base, buf, sem):
        pltpu.make_async_copy(
            x_hbm.at[pl.ds(base * C, CH * C)], buf, sem).start()

    def wait_in(buf, sem):
        pltpu.make_async_copy(
            x_hbm.at[pl.ds(0, CH * C)], buf, sem).wait()

    def start_out(obuf, base, osem):
        pltpu.make_async_copy(
            obuf, out_hbm.at[pl.ds(base * C, CH * C)], osem).start()

    def wait_out(obuf, osem):
        pltpu.make_async_copy(
            obuf, out_hbm.at[pl.ds(0, CH * C)], osem).wait()

    pltpu.sync_copy(st_hbm.at[pl.ds(16 * wid, 16)], st_v)
    s16 = st_v[...]

    for kk in range(GPW):
        r0 = s16[kk]
        r1 = s16[kk + 1]
        nrows = r1 - r0

        def zero_j(j, _):
            acc1[pl.ds(16 * j, 16)] = jnp.zeros((16,), jnp.float32)
            acc2[pl.ds(16 * j, 16)] = jnp.zeros((16,), jnp.float32)
            return 0

        lax.fori_loop(0, NSL, zero_j, 0)

        nfull = nrows >> CHS
        rem = nrows - (nfull << CHS)

        def acc_chunk(buf):
            def accj(j, _):
                off = 16 * j
                a1 = acc1[pl.ds(off, 16)]
                a2 = acc2[pl.ds(off, 16)]
                for r in range(CH):
                    x = buf[pl.ds(r * C + off, 16)]
                    a1 = a1 + x
                    a2 = a2 + x * x
                acc1[pl.ds(off, 16)] = a1
                acc2[pl.ds(off, 16)] = a2
                return 0

            lax.fori_loop(0, NSL, accj, 0)

        bufs = (bufa, bufb, bufc)
        sems = (sema, semb, semc)
        ncha = nfull + jnp.where(rem > 0, 1, 0)

        def a_base(ci):
            return jnp.minimum(r0 + ci * CH, N - CH)

        def acc_tail(buf):
            lo = r0 + nfull * CH
            shift = lo - a_base(nfull)

            def tail_r(r, _):
                ok = jnp.logical_and(r >= shift, r < shift + rem)
                fm = jnp.full((16,), jnp.where(ok, 1.0, 0.0),
                              dtype=jnp.float32)

                def tail_j(j, _):
                    sl = pl.ds(16 * j, 16)
                    x = buf[pl.ds(r * C + 16 * j, 16)] * fm
                    acc1[sl] = acc1[sl] + x
                    acc2[sl] = acc2[sl] + x * x
                    return 0

                lax.fori_loop(0, NSL, tail_j, 0)
                return 0

            lax.fori_loop(0, CH, tail_r, 0)

        @pl.when(ncha > 0)
        def _():
            start_in(a_base(0), bufa, sema)

            @pl.when(ncha > 1)
            def _():
                start_in(a_base(1), bufb, semb)

            def body(ci, p):
                for q in range(3):
                    def br(q=q):
                        wait_in(bufs[q], sems[q])

                        @pl.when(ci + 2 < ncha)
                        def _():
                            start_in(a_base(ci + 2),
                                     bufs[(q + 2) % 3], sems[(q + 2) % 3])

                        @pl.when(ci < nfull)
                        def _():
                            acc_chunk(bufs[q])

                        @pl.when(ci >= nfull)
                        def _():
                            acc_tail(bufs[q])
                    pl.when(p == q)(br)
                return jnp.where(p == 2, 0, p + 1)

            lax.fori_loop(0, ncha, body, jnp.int32(0))

        nch = (nrows + CH - 1) >> CHS
        big = nrows >= CH

        @pl.when(big)
        def _():
            start_in(r0, bufa, sema)

            @pl.when(nch > 1)
            def _():
                start_in(jnp.minimum(r0 + CH, r1 - CH), bufb, semb)

        nv = jnp.full((16,), nrows, dtype=jnp.float32)
        inv_n = 1.0 / jnp.maximum(nv, 1.0)

        def coeff_j(j, _):
            sl = pl.ds(16 * j, 16)
            m = acc1[sl] * inv_n
            ms = m * msv[sl]
            var = acc2[sl] * inv_n - 2.0 * ms * m + ms * ms
            rstd = _rsqrt_newton(var + 1e-6)
            a = wv[sl] * rstd
            av[sl] = a
            bv[sl] = biasv[sl] - a * ms
            return 0

        lax.fori_loop(0, NSL, coeff_j, 0)

        def apply_chunk(buf, obuf):
            def appj(j, _):
                off = 16 * j
                a = av[pl.ds(off, 16)]
                b = bv[pl.ds(off, 16)]
                for r in range(CH):
                    obuf[pl.ds(r * C + off, 16)] = (
                        buf[pl.ds(r * C + off, 16)] * a + b)
                return 0

            lax.fori_loop(0, NSL, appj, 0)

        @pl.when(big)
        def _():
            def body(ci, p):
                base = jnp.minimum(r0 + ci * CH, r1 - CH)
                nbase2 = jnp.minimum(r0 + (ci + 2) * CH, r1 - CH)
                even = (ci & 1) == 0
                for q in range(3):
                    def br(q=q):
                        wait_in(bufs[q], sems[q])

                        @pl.when(ci + 2 < nch)
                        def _():
                            start_in(nbase2, bufs[(q + 2) % 3],
                                     sems[(q + 2) % 3])

                        @pl.when(even)
                        def _():
                            @pl.when(ci >= 2)
                            def _():
                                wait_out(obufa, osema)
                            apply_chunk(bufs[q], obufa)
                            start_out(obufa, base, osema)

                        @pl.when(jnp.logical_not(even))
                        def _():
                            @pl.when(ci >= 2)
                            def _():
                                wait_out(obufb, osemb)
                            apply_chunk(bufs[q], obufb)
                            start_out(obufb, base, osemb)
                    pl.when(p == q)(br)
                return jnp.where(p == 2, 0, p + 1)

            lax.fori_loop(0, nch, body, jnp.int32(0))

            last_even = ((nch - 1) & 1) == 0

            @pl.when(last_even)
            def _():
                wait_out(obufa, osema)

            @pl.when(jnp.logical_not(last_even))
            def _():
                wait_out(obufb, osemb)

            @pl.when(jnp.logical_and(nch >= 2, last_even))
            def _():
                wait_out(obufb, osemb)

            @pl.when(jnp.logical_and(nch >= 2, jnp.logical_not(last_even)))
            def _():
                wait_out(obufa, osema)

        @pl.when(jnp.logical_and(nrows > 0, jnp.logical_not(big)))
        def _():
            def row_b(r, _):
                row = r0 + r
                pltpu.sync_copy(x_hbm.at[pl.ds(row * C, C)],
                                bufa.at[pl.ds(0, C)])

                def row_j(j, _):
                    obufa[pl.ds(16 * j, 16)] = (
                        bufa[pl.ds(16 * j, 16)] * av[pl.ds(16 * j, 16)]
                        + bv[pl.ds(16 * j, 16)])
                    return 0

                lax.fori_loop(0, NSL, row_j, 0)
                pltpu.sync_copy(obufa.at[pl.ds(0, C)],
                                out_hbm.at[pl.ds(row * C, C)])
                return 0

            lax.fori_loop(0, nrows, row_b, 0)


@functools.partial(jax.jit, static_argnums=(2,))
def _graph_norm(tensor, batch, num_graphs, weight, bias, mean_scale):
    del num_graphs
    x = tensor.reshape(N * C)
    bi = batch.astype(jnp.int32)
    starts = jnp.searchsorted(bi, jnp.arange(G + 1, dtype=jnp.int32),
                              method="compare_all")
    st16 = jnp.zeros((G, 16), jnp.int32)
    st16 = st16.at[:, 0].set(starts[:-1]).at[:, 1].set(starts[1:])
    st16 = st16.reshape(G * 16)

    mesh = plsc.VectorSubcoreMesh(core_axis_name="c", subcore_axis_name="s",
                                  num_cores=NC, num_subcores=NS)
    run = pl.kernel(
        _sc_body,
        out_type=jax.ShapeDtypeStruct((N * C,), jnp.float32),
        mesh=mesh,
        scratch_types=[
            pltpu.VMEM((16,), jnp.int32),
            pltpu.VMEM((CH * C,), jnp.float32),
            pltpu.VMEM((CH * C,), jnp.float32),
            pltpu.VMEM((CH * C,), jnp.float32),
            pltpu.VMEM((CH * C,), jnp.float32),
            pltpu.VMEM((CH * C,), jnp.float32),
            pltpu.VMEM((C,), jnp.float32),
            pltpu.VMEM((C,), jnp.float32),
            pltpu.VMEM((C,), jnp.float32),
            pltpu.VMEM((C,), jnp.float32),
            pltpu.VMEM((C,), jnp.float32),
            pltpu.VMEM((C,), jnp.float32),
            pltpu.VMEM((C,), jnp.float32),
            pltpu.SemaphoreType.DMA,
            pltpu.SemaphoreType.DMA,
            pltpu.SemaphoreType.DMA,
            pltpu.SemaphoreType.DMA,
            pltpu.SemaphoreType.DMA,
        ],
        compiler_params=pltpu.CompilerParams(needs_layout_passes=False),
    )
    out = run(x, st16, weight, bias, mean_scale)
    return out.reshape(N, C, 1)


def kernel(tensor, batch, num_graphs, weight, bias, mean_scale):
    return _graph_norm(tensor, batch, G, weight, bias, mean_scale)
